# Initial kernel scaffold; baseline (speedup 1.0000x reference)
import jax
import jax.numpy as jnp
from jax.experimental import pallas as pl


def kernel(x, adj_mx, mem_keys, mem_vals, Wq, bq, W1, b1, W2, b2, Wo, bo, ln_g, ln_b):
    raise NotImplementedError("write your pallas kernel here")


# speedup vs baseline: 5.0490x; 5.0490x over previous
import functools
import math

import jax
import jax.numpy as jnp
from jax.experimental import pallas as pl
from jax.experimental.pallas import tpu as pltpu

B = 32
N = 512
MEM = 16
D = 768
R = 4
TEMP = 0.7

NB = 8
T = B * NB
KM = NB * MEM
CB = 3072

_SQRT2 = math.sqrt(2.0)


def _gelu(v):
    return 0.5 * v * (1.0 + jax.lax.erf(v / _SQRT2))


def _adj_kernel(adj_ref, mk_ref, mv_ref, nk_ref, nv_ref):
    adj = adj_ref[...]
    r = jax.lax.broadcasted_iota(jnp.int32, (N, N), 0)
    c = jax.lax.broadcasted_iota(jnp.int32, (N, N), 1)
    a = adj + jnp.where(r == c, jnp.float32(1.0), jnp.float32(0.0))
    rowsum = jnp.sum(a, axis=1, keepdims=True)
    d = jnp.where(rowsum > 0, jax.lax.rsqrt(rowsum), jnp.float32(0.0))
    nk_ref[...] = jnp.dot(a, mk_ref[...] * d,
                          preferred_element_type=jnp.float32) * d
    nv_ref[...] = jnp.dot(a, mv_ref[...] * d,
                          preferred_element_type=jnp.float32) * d


def _read(qn, keys, vals):
    kn = keys * jax.lax.rsqrt(
        jnp.maximum(jnp.sum(keys * keys, axis=1, keepdims=True), 1e-24))
    sim_full = jax.lax.dot_general(qn, kn, (((1,), (1,)), ((), ())),
                                   preferred_element_type=jnp.float32)
    s = jnp.concatenate(
        [sim_full[n * B:(n + 1) * B, n * MEM:(n + 1) * MEM] for n in range(NB)],
        axis=0)
    si = s[:, :, None]
    sj = s[:, None, :]
    ii = jax.lax.broadcasted_iota(jnp.int32, (MEM, MEM), 0)[None]
    jj = jax.lax.broadcasted_iota(jnp.int32, (MEM, MEM), 1)[None]
    rank = jnp.sum(
        (sj > si).astype(jnp.float32)
        + ((sj == si) & (jj < ii)).astype(jnp.float32), axis=2)
    sel = rank < R
    m1 = jnp.max(s, axis=1, keepdims=True)
    w = jnp.where(sel, jnp.exp((s - m1) / TEMP), jnp.float32(0.0))
    w = w / jnp.sum(w, axis=1, keepdims=True)
    wb = jnp.concatenate([w] * NB, axis=1)
    rr = jax.lax.broadcasted_iota(jnp.int32, (T, KM), 0) // B
    cc = jax.lax.broadcasted_iota(jnp.int32, (T, KM), 1) // MEM
    wb = jnp.where(rr == cc, wb, jnp.float32(0.0))
    return jnp.dot(wb, vals, preferred_element_type=jnp.float32)


def _main_kernel(x_ref, mk_ref, mv_ref, nk_ref, nv_ref,
                 wq_ref, bq_ref, w1x_ref, w1s_ref, w1n_ref, b1_ref,
                 w2_ref, b2_ref, wo_ref, bo_ref, lng_ref, lnb_ref, out_ref):
    xt = jnp.transpose(x_ref[...], (1, 0, 2)).reshape(T, D)
    q = jnp.dot(xt, wq_ref[...], preferred_element_type=jnp.float32) + bq_ref[...]
    qn = q * jax.lax.rsqrt(
        jnp.maximum(jnp.sum(q * q, axis=1, keepdims=True), 1e-24))
    self_mem = _read(qn, mk_ref[...], mv_ref[...])
    nbr_mem = _read(qn, nk_ref[...], nv_ref[...])
    h = (jnp.dot(xt, w1x_ref[...], preferred_element_type=jnp.float32)
         + jnp.dot(self_mem, w1s_ref[...], preferred_element_type=jnp.float32)
         + jnp.dot(nbr_mem, w1n_ref[...], preferred_element_type=jnp.float32)
         + b1_ref[...])
    h = _gelu(h)
    fl = jnp.dot(h, w2_ref[...], preferred_element_type=jnp.float32) + b2_ref[...]
    fm = jnp.max(fl, axis=1, keepdims=True)
    fe = jnp.exp(fl - fm)
    fw = fe / jnp.sum(fe, axis=1, keepdims=True)
    fused = (fw[:, 0:1] * xt + fw[:, 1:2] * self_mem + fw[:, 2:3] * nbr_mem)
    o = _gelu(jnp.dot(fused, wo_ref[...], preferred_element_type=jnp.float32)
              + bo_ref[...])
    y = xt + o
    mu = jnp.mean(y, axis=1, keepdims=True)
    var = jnp.mean((y - mu) ** 2, axis=1, keepdims=True)
    yn = (y - mu) / jnp.sqrt(var + 1e-5) * lng_ref[...] + lnb_ref[...]
    out_ref[...] = jnp.transpose(yn.reshape(NB, B, D), (1, 0, 2))


def kernel(x, adj_mx, mem_keys, mem_vals, Wq, bq, W1, b1, W2, b2,
           Wo, bo, ln_g, ln_b):
    mk2 = mem_keys.reshape(N, MEM * D)
    mv2 = mem_vals.reshape(N, MEM * D)
    nbr_k, nbr_v = pl.pallas_call(
        _adj_kernel,
        grid=(MEM * D // CB,),
        in_specs=[
            pl.BlockSpec((N, N), lambda c: (0, 0)),
            pl.BlockSpec((N, CB), lambda c: (0, c)),
            pl.BlockSpec((N, CB), lambda c: (0, c)),
        ],
        out_specs=[
            pl.BlockSpec((N, CB), lambda c: (0, c)),
            pl.BlockSpec((N, CB), lambda c: (0, c)),
        ],
        out_shape=[
            jax.ShapeDtypeStruct((N, MEM * D), jnp.float32),
            jax.ShapeDtypeStruct((N, MEM * D), jnp.float32),
        ],
    )(adj_mx, mk2, mv2)

    mkr = mk2.reshape(N * MEM, D)
    mvr = mv2.reshape(N * MEM, D)
    nkr = nbr_k.reshape(N * MEM, D)
    nvr = nbr_v.reshape(N * MEM, D)

    full = lambda n: (0, 0)
    mem_spec = pl.BlockSpec((KM, D), lambda n: (n, 0))
    out = pl.pallas_call(
        _main_kernel,
        grid=(N // NB,),
        in_specs=[
            pl.BlockSpec((B, NB, D), lambda n: (0, n, 0)),
            mem_spec, mem_spec, mem_spec, mem_spec,
            pl.BlockSpec((D, D), full),
            pl.BlockSpec((1, D), full),
            pl.BlockSpec((D, D), full),
            pl.BlockSpec((D, D), full),
            pl.BlockSpec((D, D), full),
            pl.BlockSpec((1, D), full),
            pl.BlockSpec((D, 3), full),
            pl.BlockSpec((1, 3), full),
            pl.BlockSpec((D, D), full),
            pl.BlockSpec((1, D), full),
            pl.BlockSpec((1, D), full),
            pl.BlockSpec((1, D), full),
        ],
        out_specs=pl.BlockSpec((B, NB, D), lambda n: (0, n, 0)),
        out_shape=jax.ShapeDtypeStruct((B, N, D), jnp.float32),
    )(
        x, mkr, mvr, nkr, nvr,
        Wq.T, bq.reshape(1, D),
        W1[:, :D].T, W1[:, D:2 * D].T, W1[:, 2 * D:].T, b1.reshape(1, D),
        W2.T, b2.reshape(1, 3),
        Wo.T, bo.reshape(1, D), ln_g.reshape(1, D), ln_b.reshape(1, D),
    )
    return out
